# baseline (speedup 1.0000x reference)
import functools

import jax
import jax.numpy as jnp
from jax import lax
from jax.experimental import pallas as pl
from jax.experimental.pallas import tpu as pltpu
from jax.experimental.pallas import tpu_sc as plsc

OUT_SIZE = 12
V = OUT_SIZE * OUT_SIZE * OUT_SIZE
NC = 2
NS = 16
NW = NC * NS
L = 16
BLK = 2048
SUB = 4
UNR = 8
CAP = 96
FLUSH_AT = CAP - SUB * L

NEG_INF = float("-inf")


def _pool_body(nrois, npoints, C, n_rounds,
               prm_hbm, coords_hbm, pf_hbm, out_hbm,
               acc, cbuf, rowsf, prm, segl, pidxl, cnt_ref,
               sem0, sem1, gsem):
  acc_words = V * C
  n_vec = acc_words // L
  nblk = npoints // BLK
  assert nblk % 2 == 0
  wid = lax.axis_index("s") * NC + lax.axis_index("c")

  def blk_copy(b, buf, sem):
    return pltpu.make_async_copy(coords_hbm.at[b], cbuf.at[buf], sem)

  for t in range(n_rounds):
    r = wid + t * NW

    @pl.when(r < nrois)
    def _do_roi():
      pltpu.sync_copy(prm_hbm.at[r], prm)
      pv = prm[...]
      cx = pv[0]
      cy = pv[1]
      czc = pv[2]
      hdx = pv[3]
      hdy = pv[4]
      hdz = pv[5]
      cosa = pv[6]
      sina = pv[7]
      ixres = pv[8]
      iyres = pv[9]
      izres = pv[10]
      cz = pv[11]

      neg = jnp.full((L,), NEG_INF, jnp.float32)
      zero_i = jnp.zeros((L,), jnp.int32)

      for g in range(CAP // L):
        pidxl[pl.ds(g * L, L)] = zero_i
      cnt_ref[0] = 0

      def init_body(i, _):
        for u in range(UNR):
          acc[pl.ds((i * UNR + u) * L, L)] = neg
        return _
      lax.fori_loop(0, n_vec // UNR, init_body, None)

      def flush(n):
        pltpu.async_copy(pf_hbm.at[pidxl], rowsf, gsem).wait()
        ngr = (n + (L - 1)) // L

        def group_body(g, _):
          gb = g * L
          seg16 = segl[pl.ds(gb, L)]
          valid = jnp.where(lax.iota(jnp.int32, L) < (n - gb), 1, 0)
          for j in range(L):
            @pl.when(valid[j] > 0)
            def _upd(j=j):
              rb = seg16[j] * C
              for cb in range(C // L):
                sl = pl.ds(rb + cb * L, L)
                acc[sl] = jnp.maximum(acc[sl], rowsf[gb + j, pl.ds(cb * L, L)])
          return _
        lax.fori_loop(0, ngr, group_body, None)
        cnt_ref[0] = 0

      def scan_block(buf, base):
        def chunk_body(ci, _):
          off0 = ci * (L * SUB)
          zs_ = []
          lxs = []
          lys = []
          inbs = []
          for k in range(SUB):
            off = off0 + k * L
            x = cbuf[buf, 0, pl.ds(off, L)]
            y = cbuf[buf, 1, pl.ds(off, L)]
            z = cbuf[buf, 2, pl.ds(off, L)]
            sx = x - cx
            sy = y - cy
            zok = jnp.abs(z - czc) <= hdz
            lx = sx * cosa - sy * sina
            ly = sx * sina + sy * cosa
            inb = zok & (jnp.abs(lx) < hdx) & (jnp.abs(ly) < hdy)
            zs_.append(z)
            lxs.append(lx)
            lys.append(ly)
            inbs.append(inb)
          pk = plsc.all_reduce_population_count(inbs[0])
          for k in range(1, SUB):
            pk = pk | (plsc.all_reduce_population_count(inbs[k]) << (8 * k))
          n_all = pk[0]

          def _collect_all(_, carry):
            for k in range(SUB):
              nk = (n_all >> (8 * k)) & 0xFF

              @pl.when(nk > 0)
              def _one(lx=lxs[k], ly=lys[k], z=zs_[k], inb=inbs[k],
                       off=off0 + k * L, nk=nk):
                fx = (lx + hdx) * ixres
                fy = (ly + hdy) * iyres
                fz = (z - cz) * izres
                xi = jnp.clip(fx, 0.0, float(OUT_SIZE - 1)).astype(jnp.int32)
                yi = jnp.clip(fy, 0.0, float(OUT_SIZE - 1)).astype(jnp.int32)
                zi = jnp.clip(fz, 0.0, float(OUT_SIZE - 1)).astype(jnp.int32)
                seg = (xi * OUT_SIZE + yi) * OUT_SIZE + zi
                pidx = base + off + lax.iota(jnp.int32, L)
                n0 = cnt_ref[0]
                plsc.store_compressed(segl.at[pl.ds(n0, L)], seg, mask=inb)
                plsc.store_compressed(pidxl.at[pl.ds(n0, L)], pidx, mask=inb)
                cnt_ref[0] = n0 + nk

            @pl.when(cnt_ref[0] > FLUSH_AT)
            def _flush_now():
              flush(cnt_ref[0])
            return carry
          lax.fori_loop(0, jnp.where(n_all != 0, 1, 0), _collect_all, None)
          return _
        lax.fori_loop(0, BLK // (L * SUB), chunk_body, None)

      blk_copy(0, 0, sem0).start()

      def pair_body(bb, _):
        b0 = 2 * bb
        blk_copy(b0 + 1, 1, sem1).start()
        blk_copy(b0, 0, sem0).wait()
        scan_block(0, b0 * BLK)

        @pl.when(b0 + 2 < nblk)
        def _prefetch():
          blk_copy(b0 + 2, 0, sem0).start()
        blk_copy(b0 + 1, 1, sem1).wait()
        scan_block(1, (b0 + 1) * BLK)
        return _
      lax.fori_loop(0, nblk // 2, pair_body, None)

      @pl.when(cnt_ref[0] > 0)
      def _final_flush():
        flush(cnt_ref[0])

      def fin_body(i, _):
        for u in range(UNR):
          sl = pl.ds((i * UNR + u) * L, L)
          v = acc[sl]
          acc[sl] = jnp.where(v == NEG_INF, 0.0, v)
        return _
      lax.fori_loop(0, n_vec // UNR, fin_body, None)

      pltpu.sync_copy(acc, out_hbm.at[r])


def kernel(rois, pts, pts_feature):
  nrois = rois.shape[0]
  npoints = pts.shape[0]
  C = pts_feature.shape[1]
  assert npoints % (2 * BLK) == 0 and C % L == 0
  n_rounds = -(-nrois // NW)
  nblk = npoints // BLK

  cx, cy, cz = rois[:, 0], rois[:, 1], rois[:, 2]
  dx, dy, dz = rois[:, 3], rois[:, 4], rois[:, 5]
  rz = rois[:, 6]
  czc = cz + dz * 0.5
  cosa = jnp.cos(-rz)
  sina = jnp.sin(-rz)
  hdx, hdy, hdz = dx * 0.5, dy * 0.5, dz * 0.5
  ixres = OUT_SIZE / dx
  iyres = OUT_SIZE / dy
  izres = OUT_SIZE / dz
  pad = jnp.zeros((nrois,), jnp.float32)
  prm = jnp.stack(
      [cx, cy, czc, hdx, hdy, hdz, cosa, sina, ixres, iyres, izres, cz,
       pad, pad, pad, pad], axis=1)

  coords = jnp.transpose(pts.T.reshape(3, nblk, BLK), (1, 0, 2))

  mesh = plsc.VectorSubcoreMesh(
      core_axis_name="c", subcore_axis_name="s",
      num_cores=NC, num_subcores=NS)

  fn = pl.kernel(
      functools.partial(_pool_body, nrois, npoints, C, n_rounds),
      out_type=jax.ShapeDtypeStruct((nrois, V * C), jnp.float32),
      mesh=mesh,
      compiler_params=pltpu.CompilerParams(
          needs_layout_passes=False, use_tc_tiling_on_sc=False),
      scratch_types=[
          pltpu.VMEM((V * C,), jnp.float32),
          pltpu.VMEM((2, 3, BLK), jnp.float32),
          pltpu.VMEM((CAP, C), jnp.float32),
          pltpu.VMEM((L,), jnp.float32),
          pltpu.VMEM((CAP,), jnp.int32),
          pltpu.VMEM((CAP,), jnp.int32),
          pltpu.SMEM((1,), jnp.int32),
          pltpu.SemaphoreType.DMA,
          pltpu.SemaphoreType.DMA,
          pltpu.SemaphoreType.DMA,
      ],
  )
  out = fn(prm, coords, pts_feature)
  return out.reshape(nrois, OUT_SIZE, OUT_SIZE, OUT_SIZE, C)


# speedup vs baseline: 1.0008x; 1.0008x over previous
import functools

import jax
import jax.numpy as jnp
from jax import lax
from jax.experimental import pallas as pl
from jax.experimental.pallas import tpu as pltpu
from jax.experimental.pallas import tpu_sc as plsc

OUT_SIZE = 12
V = OUT_SIZE * OUT_SIZE * OUT_SIZE
NC = 2
NS = 16
NW = NC * NS
L = 16
BLK = 2048
SUB = 4
UNR = 8
CAP = 96
FLUSH_AT = CAP - SUB * L

NEG_INF = float("-inf")


def _pool_body(nrois, npoints, C, n_rounds,
               prm_hbm, coords_hbm, pf_hbm, out_hbm,
               acc, cbuf, rowsf, prm, segl, pidxl, cnt_ref,
               sem0, sem1, gsem):
  acc_words = V * C
  n_vec = acc_words // L
  nblk = npoints // BLK
  assert nblk % 2 == 0
  wid = lax.axis_index("s") * NC + lax.axis_index("c")

  def blk_copy(b, buf, sem):
    return pltpu.make_async_copy(coords_hbm.at[b], cbuf.at[buf], sem)

  for t in range(n_rounds):
    r = wid + t * NW

    @pl.when(r < nrois)
    def _do_roi():
      pltpu.sync_copy(prm_hbm.at[r], prm)
      pv = prm[...]
      cx = pv[0]
      cy = pv[1]
      czc = pv[2]
      hdx = pv[3]
      hdy = pv[4]
      hdz = pv[5]
      cosa = pv[6]
      sina = pv[7]
      ixres = pv[8]
      iyres = pv[9]
      izres = pv[10]
      cz = pv[11]

      neg = jnp.full((L,), NEG_INF, jnp.float32)
      zero_i = jnp.zeros((L,), jnp.int32)

      for g in range(CAP // L):
        pidxl[pl.ds(g * L, L)] = zero_i
      cnt_ref[0] = 0

      def init_body(i, _):
        for u in range(UNR):
          acc[pl.ds((i * UNR + u) * L, L)] = neg
        return _
      lax.fori_loop(0, n_vec // UNR, init_body, None)

      def flush(n):
        nq = CAP // 32
        cps = [pltpu.make_async_copy(
            pf_hbm.at[pidxl.at[pl.ds(q * 32, 32)]],
            rowsf.at[pl.ds(q * 32, 32)], gsem) for q in range(nq)]
        for cp in cps:
          cp.start()
        for cp in cps:
          cp.wait()
        ngr = (n + (L - 1)) // L

        def group_body(g, _):
          gb = g * L
          seg16 = segl[pl.ds(gb, L)]
          valid = jnp.where(lax.iota(jnp.int32, L) < (n - gb), 1, 0)
          for j in range(L):
            @pl.when(valid[j] > 0)
            def _upd(j=j):
              rb = seg16[j] * C
              for cb in range(C // L):
                sl = pl.ds(rb + cb * L, L)
                acc[sl] = jnp.maximum(acc[sl], rowsf[gb + j, pl.ds(cb * L, L)])
          return _
        lax.fori_loop(0, ngr, group_body, None)
        cnt_ref[0] = 0

      def scan_block(buf, base):
        def chunk_body(ci, _):
          off0 = ci * (L * SUB)
          zs_ = []
          lxs = []
          lys = []
          inbs = []
          for k in range(SUB):
            off = off0 + k * L
            x = cbuf[buf, 0, pl.ds(off, L)]
            y = cbuf[buf, 1, pl.ds(off, L)]
            z = cbuf[buf, 2, pl.ds(off, L)]
            sx = x - cx
            sy = y - cy
            zok = jnp.abs(z - czc) <= hdz
            lx = sx * cosa - sy * sina
            ly = sx * sina + sy * cosa
            inb = zok & (jnp.abs(lx) < hdx) & (jnp.abs(ly) < hdy)
            zs_.append(z)
            lxs.append(lx)
            lys.append(ly)
            inbs.append(inb)
          pk = plsc.all_reduce_population_count(inbs[0])
          for k in range(1, SUB):
            pk = pk | (plsc.all_reduce_population_count(inbs[k]) << (8 * k))
          n_all = pk[0]

          def _collect_all(_, carry):
            for k in range(SUB):
              nk = (n_all >> (8 * k)) & 0xFF

              @pl.when(nk > 0)
              def _one(lx=lxs[k], ly=lys[k], z=zs_[k], inb=inbs[k],
                       off=off0 + k * L, nk=nk):
                fx = (lx + hdx) * ixres
                fy = (ly + hdy) * iyres
                fz = (z - cz) * izres
                xi = jnp.clip(fx, 0.0, float(OUT_SIZE - 1)).astype(jnp.int32)
                yi = jnp.clip(fy, 0.0, float(OUT_SIZE - 1)).astype(jnp.int32)
                zi = jnp.clip(fz, 0.0, float(OUT_SIZE - 1)).astype(jnp.int32)
                seg = (xi * OUT_SIZE + yi) * OUT_SIZE + zi
                pidx = base + off + lax.iota(jnp.int32, L)
                n0 = cnt_ref[0]
                plsc.store_compressed(segl.at[pl.ds(n0, L)], seg, mask=inb)
                plsc.store_compressed(pidxl.at[pl.ds(n0, L)], pidx, mask=inb)
                cnt_ref[0] = n0 + nk

            @pl.when(cnt_ref[0] > FLUSH_AT)
            def _flush_now():
              flush(cnt_ref[0])
            return carry
          lax.fori_loop(0, jnp.where(n_all != 0, 1, 0), _collect_all, None)
          return _
        lax.fori_loop(0, BLK // (L * SUB), chunk_body, None)

      blk_copy(0, 0, sem0).start()

      def pair_body(bb, _):
        b0 = 2 * bb
        blk_copy(b0 + 1, 1, sem1).start()
        blk_copy(b0, 0, sem0).wait()
        scan_block(0, b0 * BLK)

        @pl.when(b0 + 2 < nblk)
        def _prefetch():
          blk_copy(b0 + 2, 0, sem0).start()
        blk_copy(b0 + 1, 1, sem1).wait()
        scan_block(1, (b0 + 1) * BLK)
        return _
      lax.fori_loop(0, nblk // 2, pair_body, None)

      @pl.when(cnt_ref[0] > 0)
      def _final_flush():
        flush(cnt_ref[0])

      def fin_body(i, _):
        for u in range(UNR):
          sl = pl.ds((i * UNR + u) * L, L)
          v = acc[sl]
          acc[sl] = jnp.where(v == NEG_INF, 0.0, v)
        return _
      lax.fori_loop(0, n_vec // UNR, fin_body, None)

      pltpu.sync_copy(acc, out_hbm.at[r])


def kernel(rois, pts, pts_feature):
  nrois = rois.shape[0]
  npoints = pts.shape[0]
  C = pts_feature.shape[1]
  assert npoints % (2 * BLK) == 0 and C % L == 0
  n_rounds = -(-nrois // NW)
  nblk = npoints // BLK

  cx, cy, cz = rois[:, 0], rois[:, 1], rois[:, 2]
  dx, dy, dz = rois[:, 3], rois[:, 4], rois[:, 5]
  rz = rois[:, 6]
  czc = cz + dz * 0.5
  cosa = jnp.cos(-rz)
  sina = jnp.sin(-rz)
  hdx, hdy, hdz = dx * 0.5, dy * 0.5, dz * 0.5
  ixres = OUT_SIZE / dx
  iyres = OUT_SIZE / dy
  izres = OUT_SIZE / dz
  pad = jnp.zeros((nrois,), jnp.float32)
  prm = jnp.stack(
      [cx, cy, czc, hdx, hdy, hdz, cosa, sina, ixres, iyres, izres, cz,
       pad, pad, pad, pad], axis=1)

  coords = jnp.transpose(pts.T.reshape(3, nblk, BLK), (1, 0, 2))

  mesh = plsc.VectorSubcoreMesh(
      core_axis_name="c", subcore_axis_name="s",
      num_cores=NC, num_subcores=NS)

  fn = pl.kernel(
      functools.partial(_pool_body, nrois, npoints, C, n_rounds),
      out_type=jax.ShapeDtypeStruct((nrois, V * C), jnp.float32),
      mesh=mesh,
      compiler_params=pltpu.CompilerParams(
          needs_layout_passes=False, use_tc_tiling_on_sc=False),
      scratch_types=[
          pltpu.VMEM((V * C,), jnp.float32),
          pltpu.VMEM((2, 3, BLK), jnp.float32),
          pltpu.VMEM((CAP, C), jnp.float32),
          pltpu.VMEM((L,), jnp.float32),
          pltpu.VMEM((CAP,), jnp.int32),
          pltpu.VMEM((CAP,), jnp.int32),
          pltpu.SMEM((1,), jnp.int32),
          pltpu.SemaphoreType.DMA,
          pltpu.SemaphoreType.DMA,
          pltpu.SemaphoreType.DMA,
      ],
  )
  out = fn(prm, coords, pts_feature)
  return out.reshape(nrois, OUT_SIZE, OUT_SIZE, OUT_SIZE, C)
